# initial kernel scaffold (unmeasured)
import jax
import jax.numpy as jnp
from jax import lax
from jax.experimental import pallas as pl
from jax.experimental.pallas import tpu as pltpu

N = 4096
D = 1024
B = 256
PAD = B
NB = N // B


def kernel(x, dest):
    dest = dest.astype(jnp.int32)
    order = jnp.argsort(dest, stable=True)
    x_sorted = x.astype(jnp.bfloat16)[order]
    cz = jnp.sum(dest == 0).astype(jnp.int32)

    pad = jnp.zeros((PAD, D), jnp.bfloat16)
    s_in = jnp.concatenate([pad, x_sorted, pad], axis=0)

    def body(cz_ref, s_ref, out_ref, c_ref, send_sems, recv_sems):
        my_x = lax.axis_index("x")
        my_y = lax.axis_index("y")
        my_z = lax.axis_index("z")
        partner = (my_x, 1 - my_y, my_z)
        is0 = my_y == 0

        czv = cz_ref[0]
        cov = N - czv
        nb_z = (czv + B - 1) // B
        nb_o = (cov + B - 1) // B

        barrier = pltpu.get_barrier_semaphore()
        pl.semaphore_signal(
            barrier, inc=1, device_id=partner,
            device_id_type=pl.DeviceIdType.MESH,
        )
        pl.semaphore_wait(barrier, 1)

        nb_send = jnp.where(is0, nb_o, nb_z)
        for j in range(NB):
            @pl.when(j < nb_send)
            def _():
                src0 = jnp.where(is0, PAD + N - (j + 1) * B, PAD + j * B)
                dst0 = jnp.where(is0, PAD + cov - (j + 1) * B, PAD + cov + j * B)
                rdma = pltpu.make_async_remote_copy(
                    src_ref=s_ref.at[pl.ds(src0, B)],
                    dst_ref=c_ref.at[pl.ds(dst0, B)],
                    send_sem=send_sems.at[j],
                    recv_sem=recv_sems.at[j],
                    device_id=partner,
                    device_id_type=pl.DeviceIdType.MESH,
                )
                rdma.start()

        nb_keep = jnp.where(is0, nb_z, nb_o)
        for j in range(NB):
            @pl.when(j < nb_keep)
            def _():
                start = jnp.where(
                    is0, PAD + czv - (j + 1) * B, PAD + czv + j * B
                )
                c_ref[pl.ds(start, B), :] = s_ref[pl.ds(start, B), :]

        nb_recv = jnp.where(is0, nb_o, nb_z)
        for j in range(NB):
            @pl.when(j < nb_recv)
            def _():
                dst0 = jnp.where(
                    is0, PAD + czv + j * B, PAD + czv - (j + 1) * B
                )
                rdma = pltpu.make_async_remote_copy(
                    src_ref=s_ref.at[pl.ds(0, B)],
                    dst_ref=c_ref.at[pl.ds(dst0, B)],
                    send_sem=send_sems.at[j],
                    recv_sem=recv_sems.at[j],
                    device_id=partner,
                    device_id_type=pl.DeviceIdType.MESH,
                )
                rdma.wait_recv()

        for j in range(NB):
            @pl.when(j < nb_send)
            def _():
                rdma = pltpu.make_async_remote_copy(
                    src_ref=s_ref.at[pl.ds(0, B)],
                    dst_ref=c_ref.at[pl.ds(0, B)],
                    send_sem=send_sems.at[j],
                    recv_sem=recv_sems.at[j],
                    device_id=partner,
                    device_id_type=pl.DeviceIdType.MESH,
                )
                rdma.wait_send()

        out_ref[:, :] = c_ref[PAD:PAD + N, :]

    return pl.pallas_call(
        body,
        out_shape=jax.ShapeDtypeStruct((N, D), jnp.bfloat16),
        in_specs=[
            pl.BlockSpec(memory_space=pltpu.SMEM),
            pl.BlockSpec(memory_space=pltpu.VMEM),
        ],
        out_specs=pl.BlockSpec(memory_space=pltpu.VMEM),
        scratch_shapes=[
            pltpu.VMEM((N + 2 * PAD, D), jnp.bfloat16),
            pltpu.SemaphoreType.DMA((NB,)),
            pltpu.SemaphoreType.DMA((NB,)),
        ],
        compiler_params=pltpu.CompilerParams(collective_id=0),
    )(cz.reshape((1,)), s_in)


# baseline (device time: 124809 ns/iter reference)
import jax
import jax.numpy as jnp
from jax import lax
from jax.experimental import pallas as pl
from jax.experimental.pallas import tpu as pltpu

N = 4096
D = 1024
B = 256
PADC = B
NBS = N // B + 1


def kernel(x, dest):
    dest = dest.astype(jnp.int32)
    my_y = lax.axis_index("y")
    is0 = my_y == 0

    x_bf = x.astype(jnp.bfloat16)

    isz = dest == 0
    czc = jnp.cumsum(isz.astype(jnp.int32))
    coc = jnp.cumsum(1 - isz.astype(jnp.int32))
    cz = czc[N - 1]
    inv = jnp.where(isz, czc - 1, cz + coc - 1)
    perm = jnp.zeros((N,), jnp.int32).at[inv].set(
        jnp.arange(N, dtype=jnp.int32)
    )
    sorted_x = x_bf[perm]

    s = jnp.where(is0, N - cz, cz)
    sh = jnp.where(is0, 0, (N - s) % 8)
    p = jnp.arange(N + B, dtype=jnp.int32)
    src_idx = jnp.where(is0, cz + p, p - sh)
    buf_send = sorted_x[jnp.clip(src_idx, 0, N - 1)]

    def body(cz_ref, sorted_ref, send_ref, out_ref, c_ref, send_sems, recv_sems):
        my_x = lax.axis_index("x")
        my_yv = lax.axis_index("y")
        my_z = lax.axis_index("z")
        partner = (my_x, 1 - my_yv, my_z)
        i0 = my_yv == 0

        czv = cz_ref[0]
        sv = jnp.where(i0, N - czv, czv)
        kv = N - sv
        sh_snd = jnp.where(i0, 0, (N - sv) % 8)
        dst_base = jnp.where(i0, PADC, PADC + (N - sv) - sh_snd)
        dst_base = pl.multiple_of(dst_base, 8)
        nb_snd = (sh_snd + sv + B - 1) // B
        sh_rcv = jnp.where(i0, kv % 8, 0)
        rcv_base = jnp.where(i0, PADC + kv - sh_rcv, PADC)
        rcv_base = pl.multiple_of(rcv_base, 8)
        nb_rcv = (sh_rcv + sv + B - 1) // B

        barrier = pltpu.get_barrier_semaphore()
        pl.semaphore_signal(
            barrier, inc=1, device_id=partner,
            device_id_type=pl.DeviceIdType.MESH,
        )
        pl.semaphore_wait(barrier, 1)

        for j in range(NBS):
            @pl.when(j < nb_snd)
            def _():
                rdma = pltpu.make_async_remote_copy(
                    src_ref=send_ref.at[pl.ds(j * B, B)],
                    dst_ref=c_ref.at[pl.ds(dst_base + j * B, B)],
                    send_sem=send_sems.at[j],
                    recv_sem=recv_sems.at[j],
                    device_id=partner,
                    device_id_type=pl.DeviceIdType.MESH,
                )
                rdma.start()

        for j in range(NBS):
            @pl.when(j < nb_rcv)
            def _():
                rdma = pltpu.make_async_remote_copy(
                    src_ref=send_ref.at[pl.ds(0, B)],
                    dst_ref=c_ref.at[pl.ds(rcv_base + j * B, B)],
                    send_sem=send_sems.at[j],
                    recv_sem=recv_sems.at[j],
                    device_id=partner,
                    device_id_type=pl.DeviceIdType.MESH,
                )
                rdma.wait_recv()

        lo = jnp.where(i0, 0, sv)
        hi = jnp.where(i0, kv, N)
        for j in range(N // B):
            row = j * B + lax.broadcasted_iota(jnp.int32, (B, 1), 0)
            take_sorted = (row >= lo) & (row < hi)
            out_ref[j * B:(j + 1) * B, :] = jnp.where(
                take_sorted,
                sorted_ref[j * B:(j + 1) * B, :],
                c_ref[PADC + j * B:PADC + (j + 1) * B, :],
            )

        for j in range(NBS):
            @pl.when(j < nb_snd)
            def _():
                rdma = pltpu.make_async_remote_copy(
                    src_ref=send_ref.at[pl.ds(0, B)],
                    dst_ref=c_ref.at[pl.ds(0, B)],
                    send_sem=send_sems.at[j],
                    recv_sem=recv_sems.at[j],
                    device_id=partner,
                    device_id_type=pl.DeviceIdType.MESH,
                )
                rdma.wait_send()

    return pl.pallas_call(
        body,
        out_shape=jax.ShapeDtypeStruct((N, D), jnp.bfloat16),
        in_specs=[
            pl.BlockSpec(memory_space=pltpu.SMEM),
            pl.BlockSpec(memory_space=pltpu.VMEM),
            pl.BlockSpec(memory_space=pltpu.VMEM),
        ],
        out_specs=pl.BlockSpec(memory_space=pltpu.VMEM),
        scratch_shapes=[
            pltpu.VMEM((N + 2 * B, D), jnp.bfloat16),
            pltpu.SemaphoreType.DMA((NBS,)),
            pltpu.SemaphoreType.DMA((NBS,)),
        ],
        compiler_params=pltpu.CompilerParams(collective_id=0),
    )(cz.reshape((1,)), sorted_x, buf_send)
